# baseline (speedup 1.0000x reference)
import jax
import jax.numpy as jnp
from jax import lax
from jax.experimental import pallas as pl
from jax.experimental.pallas import tpu as pltpu
from jax.experimental.pallas import tpu_sc as plsc

N_NODES = 10000
D = 128
DH = D // 2
N_EDGES = 320000

NPAD = 10240
NC, NS = 2, 16
NW = NC * NS
E_PER_TILE = N_EDGES // NW
K = 128
NCHUNK = 160
EPT_PAD = NCHUNK * K
NBUF = 4
STAG = 2
RPT = NPAD // NS

_mesh = plsc.VectorSubcoreMesh(core_axis_name="c", subcore_axis_name="s")


def _deg_body(dst_hbm, out_hbm, didx_v, deg_v, row_v, res_v, stage_sh):
    cid = lax.axis_index("c")
    sid = lax.axis_index("s")
    g = cid * NS + sid

    zeros16 = jnp.zeros((16,), jnp.float32)

    def zero_deg(i, carry):
        deg_v[pl.ds(i * 16, 16)] = zeros16
        return carry

    lax.fori_loop(0, NPAD // 16, zero_deg, 0)

    pltpu.sync_copy(dst_hbm.at[pl.ds(g * E_PER_TILE, E_PER_TILE)], didx_v)

    ones16 = jnp.ones((16,), jnp.float32)

    def acc_body(i, carry):
        idx = didx_v[pl.ds(i * 16, 16)]
        plsc.addupdate_scatter(deg_v, [idx], ones16)
        return carry

    lax.fori_loop(0, E_PER_TILE // 16, acc_body, 0)

    pltpu.sync_copy(deg_v, stage_sh.at[sid])
    plsc.subcore_barrier()

    def zero_res(i, carry):
        res_v[pl.ds(i * 16, 16)] = zeros16
        return carry

    lax.fori_loop(0, RPT // 16, zero_res, 0)

    for r in range(NS):
        pltpu.sync_copy(stage_sh.at[r, pl.ds(sid * RPT, RPT)], row_v)

        def add_body(ci, carry):
            sl = pl.ds(ci * 16, 16)
            res_v[sl] = res_v[sl] + row_v[sl]
            return carry

        lax.fori_loop(0, RPT // 16, add_body, 0)

    pltpu.sync_copy(res_v, out_hbm.at[cid, pl.ds(sid * RPT, RPT)])


_deg_kernel = pl.kernel(
    _deg_body,
    out_type=jax.ShapeDtypeStruct((NC, NPAD), jnp.float32),
    mesh=_mesh,
    scratch_types=[
        pltpu.VMEM((E_PER_TILE,), jnp.int32),
        pltpu.VMEM((NPAD,), jnp.float32),
        pltpu.VMEM((RPT,), jnp.float32),
        pltpu.VMEM((RPT,), jnp.float32),
        pltpu.VMEM_SHARED((NS, NPAD), jnp.float32),
    ],
    compiler_params=pltpu.CompilerParams(needs_layout_passes=False),
)


def _agg_body(y_hbm, src_hbm, dst_hbm, out_hbm, sidx_v, didx_v, rows_v, acc_sh,
              *sems):
    gsems = sems[:NBUF]
    ssems = sems[NBUF:]
    cid = lax.axis_index("c")
    sid = lax.axis_index("s")
    rbase = sid * RPT

    pltpu.sync_copy(src_hbm.at[cid, sid], sidx_v)
    pltpu.sync_copy(dst_hbm.at[sid], didx_v)

    buf0 = rows_v.at[0]
    for b in range(RPT // K):
        sl = pl.ds(rbase + b * K, K)
        pltpu.sync_copy(y_hbm.at[pl.ds(cid * NPAD + rbase + b * K, K)], buf0)
        pltpu.sync_copy(buf0, acc_sh.at[sl])

    plsc.subcore_barrier()

    def g_issue(j, b):
        pltpu.async_copy(y_hbm.at[sidx_v.at[j]], rows_v.at[b], gsems[b])

    def g_wait(b):
        pltpu.make_async_copy(y_hbm.at[sidx_v.at[0]], rows_v.at[b],
                              gsems[b]).wait()

    def s_issue(j, b):
        pltpu.async_copy(rows_v.at[b], acc_sh.at[didx_v.at[j]], ssems[b],
                         add=True)

    def s_wait(b):
        pltpu.make_async_copy(rows_v.at[b], acc_sh.at[didx_v.at[0]],
                              ssems[b]).wait()

    g_issue(0, 0)
    g_issue(1, 1)
    g_issue(2, 2)
    g_wait(0)
    s_issue(0, 0)
    g_issue(3, 3)
    g_wait(1)
    s_issue(1, 1)

    def group_body(gi, carry):
        for b in range(NBUF):
            j = gi * NBUF + b
            s_wait(b)
            g_issue(j, b)
            bw = (b + STAG) % NBUF
            g_wait(bw)
            s_issue(j - STAG, bw)
        return carry

    lax.fori_loop(1, NCHUNK // NBUF, group_body, 0)

    g_wait((NCHUNK - STAG) % NBUF)
    s_issue(NCHUNK - STAG, (NCHUNK - STAG) % NBUF)
    g_wait((NCHUNK - 1) % NBUF)
    s_issue(NCHUNK - 1, (NCHUNK - 1) % NBUF)
    for b in range(NBUF):
        s_wait(b)

    plsc.subcore_barrier()

    for b in range(RPT // K):
        sl = pl.ds(rbase + b * K, K)
        pltpu.sync_copy(acc_sh.at[sl], buf0)
        pltpu.sync_copy(buf0, out_hbm.at[cid, sl])


_agg_kernel = pl.kernel(
    _agg_body,
    out_type=jax.ShapeDtypeStruct((NC, NPAD, DH), jnp.float32),
    mesh=_mesh,
    scratch_types=[
        pltpu.VMEM((NCHUNK, K), jnp.int32),
        pltpu.VMEM((NCHUNK, K), jnp.int32),
        pltpu.VMEM((NBUF, K, DH), jnp.float32),
        pltpu.VMEM_SHARED((NPAD, DH), jnp.float32),
    ]
    + [pltpu.SemaphoreType.DMA] * (2 * NBUF),
    compiler_params=pltpu.CompilerParams(use_tc_tiling_on_sc=False),
)


BLK = 1024
GRID = NPAD // BLK


def _mm_body(x_ref, w_ref, o_ref):
    o_ref[...] = jnp.dot(x_ref[...], w_ref[...], preferred_element_type=jnp.float32)


def _tc_matmul(x, w):
    return pl.pallas_call(
        _mm_body,
        grid=(GRID,),
        in_specs=[
            pl.BlockSpec((BLK, D), lambda i: (i, 0)),
            pl.BlockSpec((D, D), lambda i: (0, 0)),
        ],
        out_specs=pl.BlockSpec((BLK, D), lambda i: (i, 0)),
        out_shape=jax.ShapeDtypeStruct((NPAD, D), jnp.float32),
    )(x, w)


def _scale_body(degT_ref, xw_ref, y_ref, dinv_ref):
    d = degT_ref[...]
    dinv = lax.rsqrt(d[:, 0:1] + d[:, 1:2] + 1.0)
    dinv_ref[...] = dinv
    y = xw_ref[...] * dinv
    y_ref[0] = y[:, :DH]
    y_ref[1] = y[:, DH:]


def _tc_scale(degT, xw):
    return pl.pallas_call(
        _scale_body,
        grid=(GRID,),
        in_specs=[
            pl.BlockSpec((BLK, 2), lambda i: (i, 0)),
            pl.BlockSpec((BLK, D), lambda i: (i, 0)),
        ],
        out_specs=[
            pl.BlockSpec((NC, BLK, DH), lambda i: (0, i, 0)),
            pl.BlockSpec((BLK, 1), lambda i: (i, 0)),
        ],
        out_shape=[
            jax.ShapeDtypeStruct((NC, NPAD, DH), jnp.float32),
            jax.ShapeDtypeStruct((NPAD, 1), jnp.float32),
        ],
    )(degT, xw)


def _mid_body(p0_ref, p1_ref, dinv_ref, b1_ref, w2_ref, y2_ref):
    dinv = dinv_ref[...]
    ph = jnp.concatenate([p0_ref[...], p1_ref[...]], axis=1)
    h = jnp.maximum(ph * dinv + b1_ref[...], 0.0)
    y2 = jnp.dot(h, w2_ref[...], preferred_element_type=jnp.float32) * dinv
    y2_ref[0] = y2[:, :DH]
    y2_ref[1] = y2[:, DH:]


def _tc_mid(p0, p1, dinv, b1, w2):
    return pl.pallas_call(
        _mid_body,
        grid=(GRID,),
        in_specs=[
            pl.BlockSpec((BLK, DH), lambda i: (i, 0)),
            pl.BlockSpec((BLK, DH), lambda i: (i, 0)),
            pl.BlockSpec((BLK, 1), lambda i: (i, 0)),
            pl.BlockSpec((1, D), lambda i: (0, 0)),
            pl.BlockSpec((D, D), lambda i: (0, 0)),
        ],
        out_specs=pl.BlockSpec((NC, BLK, DH), lambda i: (0, i, 0)),
        out_shape=jax.ShapeDtypeStruct((NC, NPAD, DH), jnp.float32),
    )(p0, p1, dinv, b1, w2)


def _out_body(q0_ref, q1_ref, dinv_ref, b2_ref, o_ref):
    q = jnp.concatenate([q0_ref[...], q1_ref[...]], axis=1)
    o_ref[...] = q * dinv_ref[...] + b2_ref[...]


def _tc_out(q0, q1, dinv, b2):
    return pl.pallas_call(
        _out_body,
        grid=(GRID,),
        in_specs=[
            pl.BlockSpec((BLK, DH), lambda i: (i, 0)),
            pl.BlockSpec((BLK, DH), lambda i: (i, 0)),
            pl.BlockSpec((BLK, 1), lambda i: (i, 0)),
            pl.BlockSpec((1, D), lambda i: (0, 0)),
        ],
        out_specs=pl.BlockSpec((BLK, D), lambda i: (i, 0)),
        out_shape=jax.ShapeDtypeStruct((NPAD, D), jnp.float32),
    )(q0, q1, dinv, b2)


def kernel(x, edge_index, W1, b1, W2, b2):
    src = edge_index[0].astype(jnp.int32)
    dst = edge_index[1].astype(jnp.int32)
    xp = jnp.pad(x, ((0, NPAD - N_NODES), (0, 0)))

    e_pad = NS * EPT_PAD - N_EDGES
    srcp = jnp.concatenate([src, jnp.zeros((e_pad,), jnp.int32)])
    src4 = (srcp[None, :] + jnp.array([0, NPAD], jnp.int32)[:, None])
    src4 = src4.reshape(NC, NS, NCHUNK, K)
    dst3 = jnp.concatenate([dst, jnp.full((e_pad,), NPAD - 1, jnp.int32)])
    dst3 = dst3.reshape(NS, NCHUNK, K)

    deg = _deg_kernel(dst)
    xw1 = _tc_matmul(xp, W1)
    y1, dinv = _tc_scale(deg.T, xw1)

    p = _agg_kernel(y1.reshape(NC * NPAD, DH), src4, dst3)
    y2 = _tc_mid(p[0], p[1], dinv, b1.reshape(1, D), W2)

    q = _agg_kernel(y2.reshape(NC * NPAD, DH), src4, dst3)
    out = _tc_out(q[0], q[1], dinv, b2.reshape(1, D))
    return out[:N_NODES]
